# baseline (speedup 1.0000x reference)
import functools

import jax
import jax.numpy as jnp
from jax import lax
from jax.experimental import pallas as pl
from jax.experimental.pallas import tpu as pltpu
from jax.experimental.pallas import tpu_sc as plsc

_N = 16384
_C = 10000
_NC = 2
_NS = 16
_NW = _NC * _NS
_PW = _N // _NW
_CHUNK = 128
_NCH = _PW // _CHUNK
_L = 16


def _body(prob_hbm, tgt_hbm, rew_hbm, out_hbm,
          tgt_v, idx_v, val_v, rew_v, acc_v,
          t0sem, t1sem, rsem,
          g0sem, g1sem, g2sem, g3sem, g4sem, g5sem, g6sem, g7sem):
    cid = lax.axis_index("c")
    sid = lax.axis_index("s")
    wid = sid * _NC + cid
    base = wid * _PW

    tcopy0 = pltpu.async_copy(tgt_hbm.at[pl.ds(base, _CHUNK)],
                              tgt_v.at[pl.ds(0, _CHUNK)], t0sem)
    tcopy1 = pltpu.async_copy(tgt_hbm.at[pl.ds(base + _CHUNK, _PW - _CHUNK)],
                              tgt_v.at[pl.ds(_CHUNK, _PW - _CHUNK)], t1sem)
    rcopy = pltpu.async_copy(rew_hbm.at[pl.ds(base, _PW)], rew_v, rsem)

    lane = lax.iota(jnp.int32, _L)
    gsems = [g0sem, g1sem, g2sem, g3sem, g4sem, g5sem, g6sem, g7sem]
    gathers = []
    widr = lax.shift_left(wid, 12)
    tcopy0.wait()
    for j in range(_NCH):
        if j == 1:
            tcopy1.wait()
        for g in range(_CHUNK // _L):
            off = j * _CHUNK + g * _L
            rconst = ((off >> 7) << 10) + (off & 127)
            c = tgt_v[pl.ds(off, _L)]
            k = (
                lax.shift_left(lax.bitwise_and(c, jnp.int32(-8)), 14)
                + lax.shift_left(lax.bitwise_and(c, 7), 7)
                + (widr + (rconst + lane))
            )
            idx_v[j, pl.ds(g * _L, _L)] = k
        gathers.append(
            pltpu.async_copy(
                prob_hbm.at[idx_v.at[j]],
                val_v.at[pl.ds(j * _CHUNK, _CHUNK)],
                gsems[j],
            )
        )

    rcopy.wait()
    acc0 = jnp.zeros((_L,), jnp.float32)
    acc1 = jnp.zeros((_L,), jnp.float32)
    for j in range(_NCH):
        gathers[j].wait()
        for g in range(0, _CHUNK // _L, 2):
            t = j * (_CHUNK // _L) + g
            acc0 = acc0 + val_v[pl.ds(t * _L, _L)] * rew_v[pl.ds(t * _L, _L)]
            acc1 = acc1 + val_v[pl.ds((t + 1) * _L, _L)] * rew_v[pl.ds((t + 1) * _L, _L)]
    acc_v[...] = (acc0 + acc1) * (-1.0 / _N)
    pltpu.sync_copy(acc_v, out_hbm.at[wid])


_gather_loss = functools.partial(
    pl.kernel,
    out_type=jax.ShapeDtypeStruct((_NW, _L), jnp.float32),
    mesh=plsc.VectorSubcoreMesh(core_axis_name="c", subcore_axis_name="s",
                                num_cores=2),
    scratch_types=[
        pltpu.VMEM((_PW,), jnp.int32),
        pltpu.VMEM((_NCH, _CHUNK), jnp.int32),
        pltpu.VMEM((_PW,), jnp.float32),
        pltpu.VMEM((_PW,), jnp.float32),
        pltpu.VMEM((_L,), jnp.float32),
        pltpu.SemaphoreType.DMA,
        pltpu.SemaphoreType.DMA,
        pltpu.SemaphoreType.DMA,
        pltpu.SemaphoreType.DMA,
        pltpu.SemaphoreType.DMA,
        pltpu.SemaphoreType.DMA,
        pltpu.SemaphoreType.DMA,
        pltpu.SemaphoreType.DMA,
        pltpu.SemaphoreType.DMA,
        pltpu.SemaphoreType.DMA,
        pltpu.SemaphoreType.DMA,
    ],
)(_body)


def kernel(prob, targets, reward):
    pflat = prob.reshape(128, 128, 1250, 8).transpose(2, 0, 3, 1).reshape(-1)
    part = _gather_loss(pflat, targets, reward)
    return jnp.sum(part)


# speedup vs baseline: 1.0252x; 1.0252x over previous
import functools

import jax
import jax.numpy as jnp
from jax import lax
from jax.experimental import pallas as pl
from jax.experimental.pallas import tpu as pltpu
from jax.experimental.pallas import tpu_sc as plsc

_N = 16384
_C = 10000
_NC = 1
_NS = 16
_NW = _NC * _NS
_PW = _N // _NW
_CHUNK = 128
_NCH = _PW // _CHUNK
_L = 16


def _body(prob_hbm, tgt_hbm, rew_hbm, out_hbm,
          tgt_v, idx_v, val_v, rew_v, acc_v,
          t0sem, t1sem, t2sem, rsem,
          g0sem, g1sem, g2sem, g3sem, g4sem, g5sem, g6sem, g7sem):
    cid = lax.axis_index("c")
    sid = lax.axis_index("s")
    wid = sid * _NC + cid
    base = wid * _PW

    tcopy0 = pltpu.async_copy(tgt_hbm.at[pl.ds(base, _CHUNK)],
                              tgt_v.at[pl.ds(0, _CHUNK)], t0sem)
    tcopy1 = pltpu.async_copy(tgt_hbm.at[pl.ds(base + _CHUNK, _CHUNK)],
                              tgt_v.at[pl.ds(_CHUNK, _CHUNK)], t1sem)
    tcopy2 = pltpu.async_copy(tgt_hbm.at[pl.ds(base + 2 * _CHUNK, _PW - 2 * _CHUNK)],
                              tgt_v.at[pl.ds(2 * _CHUNK, _PW - 2 * _CHUNK)], t2sem)
    rcopy = pltpu.async_copy(rew_hbm.at[pl.ds(base, _PW)], rew_v, rsem)

    lane = lax.iota(jnp.int32, _L)
    gsems = [g0sem, g1sem, g2sem, g3sem, g4sem, g5sem, g6sem, g7sem]
    gathers = []
    widr = lax.shift_left(wid, 13)
    tcopy0.wait()
    for j in range(_NCH):
        if j == 1:
            tcopy1.wait()
        if j == 2:
            tcopy2.wait()
        for g in range(_CHUNK // _L):
            off = j * _CHUNK + g * _L
            rconst = ((off >> 7) << 10) + (off & 127)
            c = tgt_v[pl.ds(off, _L)]
            k = (
                lax.shift_left(lax.bitwise_and(c, jnp.int32(-8)), 14)
                + lax.shift_left(lax.bitwise_and(c, 7), 7)
                + (widr + (rconst + lane))
            )
            idx_v[j, pl.ds(g * _L, _L)] = k
        gathers.append(
            pltpu.async_copy(
                prob_hbm.at[idx_v.at[j]],
                val_v.at[pl.ds(j * _CHUNK, _CHUNK)],
                gsems[j],
            )
        )

    rcopy.wait()
    acc0 = jnp.zeros((_L,), jnp.float32)
    acc1 = jnp.zeros((_L,), jnp.float32)
    for j in range(_NCH):
        gathers[j].wait()
        for g in range(0, _CHUNK // _L, 2):
            t = j * (_CHUNK // _L) + g
            acc0 = acc0 + val_v[pl.ds(t * _L, _L)] * rew_v[pl.ds(t * _L, _L)]
            acc1 = acc1 + val_v[pl.ds((t + 1) * _L, _L)] * rew_v[pl.ds((t + 1) * _L, _L)]
    acc_v[...] = (acc0 + acc1) * (-1.0 / _N)
    pltpu.sync_copy(acc_v, out_hbm.at[wid])


_gather_loss = functools.partial(
    pl.kernel,
    out_type=jax.ShapeDtypeStruct((_NW, _L), jnp.float32),
    mesh=plsc.VectorSubcoreMesh(core_axis_name="c", subcore_axis_name="s",
                                num_cores=1),
    scratch_types=[
        pltpu.VMEM((_PW,), jnp.int32),
        pltpu.VMEM((_NCH, _CHUNK), jnp.int32),
        pltpu.VMEM((_PW,), jnp.float32),
        pltpu.VMEM((_PW,), jnp.float32),
        pltpu.VMEM((_L,), jnp.float32),
        pltpu.SemaphoreType.DMA,
        pltpu.SemaphoreType.DMA,
        pltpu.SemaphoreType.DMA,
        pltpu.SemaphoreType.DMA,
        pltpu.SemaphoreType.DMA,
        pltpu.SemaphoreType.DMA,
        pltpu.SemaphoreType.DMA,
        pltpu.SemaphoreType.DMA,
        pltpu.SemaphoreType.DMA,
        pltpu.SemaphoreType.DMA,
        pltpu.SemaphoreType.DMA,
        pltpu.SemaphoreType.DMA,
    ],
)(_body)


def kernel(prob, targets, reward):
    pflat = prob.reshape(128, 128, 1250, 8).transpose(2, 0, 3, 1).reshape(-1)
    part = _gather_loss(pflat, targets, reward)
    return jnp.sum(part)


# speedup vs baseline: 1.0718x; 1.0454x over previous
import functools

import jax
import jax.numpy as jnp
from jax import lax
from jax.experimental import pallas as pl
from jax.experimental.pallas import tpu as pltpu
from jax.experimental.pallas import tpu_sc as plsc

_N = 16384
_C = 10000
_NC = 1
_NS = 16
_NW = _NC * _NS
_PW = _N // _NW
_CHUNK = 128
_NCH = _PW // _CHUNK
_L = 16


def _body(prob_hbm, tgt_hbm, rew_hbm, out_hbm,
          tgt_v, idx_v, val_v, rew_v, acc_v, zero_v, izero_v, idx1_v,
          iz_sh, red_sh, lt_v, lt2_v,
          t0sem, t1sem, rsem,
          g0sem, g1sem, g2sem, g3sem, g4sem, g5sem, g6sem, g7sem):
    cid = lax.axis_index("c")
    sid = lax.axis_index("s")
    wid = sid * _NC + cid
    base = wid * _PW

    tcopy0 = pltpu.async_copy(tgt_hbm.at[pl.ds(base, _CHUNK)],
                              tgt_v.at[pl.ds(0, _CHUNK)], t0sem)
    tcopy1 = pltpu.async_copy(tgt_hbm.at[pl.ds(base + _CHUNK, _PW - _CHUNK)],
                              tgt_v.at[pl.ds(_CHUNK, _PW - _CHUNK)], t1sem)
    rcopy = pltpu.async_copy(rew_hbm.at[pl.ds(base, _PW)], rew_v, rsem)

    lane = lax.iota(jnp.int32, _L)
    gsems = [g0sem, g1sem, g2sem, g3sem, g4sem, g5sem, g6sem, g7sem]
    gathers = []
    widr = lax.shift_left(wid, 13)
    tcopy0.wait()
    for j in range(_NCH):
        if j == 1:
            tcopy1.wait()
        for g in range(_CHUNK // _L):
            off = j * _CHUNK + g * _L
            rconst = ((off >> 7) << 10) + (off & 127)
            c = tgt_v[pl.ds(off, _L)]
            k = (
                lax.shift_left(lax.bitwise_and(c, jnp.int32(-8)), 14)
                + lax.shift_left(lax.bitwise_and(c, 7), 7)
                + (widr + (rconst + lane))
            )
            idx_v[j, pl.ds(g * _L, _L)] = k
        gathers.append(
            pltpu.async_copy(
                prob_hbm.at[idx_v.at[j]],
                val_v.at[pl.ds(j * _CHUNK, _CHUNK)],
                gsems[j],
            )
        )

    rcopy.wait()
    acc0 = jnp.zeros((_L,), jnp.float32)
    acc1 = jnp.zeros((_L,), jnp.float32)
    for j in range(_NCH):
        gathers[j].wait()
        for g in range(0, _CHUNK // _L, 2):
            t = j * (_CHUNK // _L) + g
            acc0 = acc0 + val_v[pl.ds(t * _L, _L)] * rew_v[pl.ds(t * _L, _L)]
            acc1 = acc1 + val_v[pl.ds((t + 1) * _L, _L)] * rew_v[pl.ds((t + 1) * _L, _L)]
    acc_v[0, :] = acc0 + acc1

    izero_v[...] = jnp.zeros((_L,), jnp.int32)
    pltpu.sync_copy(izero_v, iz_sh.at[sid])
    pltpu.sync_copy(iz_sh.at[sid].at[pl.ds(0, 1)], idx1_v)

    @pl.when(sid == 0)
    def _():
        zero_v[...] = jnp.zeros((_L,), jnp.float32)
        pltpu.sync_copy(zero_v, red_sh.at[0])
    plsc.subcore_barrier()
    pltpu.sync_copy(acc_v, red_sh.at[idx1_v], add=True)
    plsc.subcore_barrier()

    @pl.when(sid == 0)
    def _():
        pltpu.sync_copy(red_sh.at[0], lt_v)
        tot = lt_v[...]
        lt2_v[pl.ds(_L, _L)] = tot
        for s in (8, 4, 2, 1):
            lt2_v[pl.ds(0, _L)] = tot
            tot = tot + lt2_v[pl.ds(s, _L)]
        zero_v[...] = tot * (-1.0 / _N)
        pltpu.sync_copy(zero_v.at[pl.ds(0, 1)], out_hbm)


_gather_loss = functools.partial(
    pl.kernel,
    out_type=jax.ShapeDtypeStruct((1,), jnp.float32),
    mesh=plsc.VectorSubcoreMesh(core_axis_name="c", subcore_axis_name="s",
                                num_cores=1),
    scratch_types=[
        pltpu.VMEM((_PW,), jnp.int32),
        pltpu.VMEM((_NCH, _CHUNK), jnp.int32),
        pltpu.VMEM((_PW,), jnp.float32),
        pltpu.VMEM((_PW,), jnp.float32),
        pltpu.VMEM((1, _L), jnp.float32),
        pltpu.VMEM((_L,), jnp.float32),
        pltpu.VMEM((_L,), jnp.int32),
        pltpu.VMEM((1,), jnp.int32),
        pltpu.VMEM_SHARED((_NS, _L), jnp.int32),
        pltpu.VMEM_SHARED((1, _L), jnp.float32),
        pltpu.VMEM((_L,), jnp.float32),
        pltpu.VMEM((2 * _L,), jnp.float32),
        pltpu.SemaphoreType.DMA,
        pltpu.SemaphoreType.DMA,
        pltpu.SemaphoreType.DMA,
        pltpu.SemaphoreType.DMA,
        pltpu.SemaphoreType.DMA,
        pltpu.SemaphoreType.DMA,
        pltpu.SemaphoreType.DMA,
        pltpu.SemaphoreType.DMA,
        pltpu.SemaphoreType.DMA,
        pltpu.SemaphoreType.DMA,
        pltpu.SemaphoreType.DMA,
    ],
)(_body)


def kernel(prob, targets, reward):
    pflat = prob.reshape(128, 128, 1250, 8).transpose(2, 0, 3, 1).reshape(-1)
    return jnp.reshape(_gather_loss(pflat, targets, reward), ())


# speedup vs baseline: 1.0756x; 1.0035x over previous
import functools

import jax
import jax.numpy as jnp
from jax import lax
from jax.experimental import pallas as pl
from jax.experimental.pallas import tpu as pltpu
from jax.experimental.pallas import tpu_sc as plsc

_N = 16384
_C = 10000
_NC = 1
_NS = 16
_NW = _NC * _NS
_PW = _N // _NW
_CHUNK = 128
_NCH = _PW // _CHUNK
_L = 16


def _body(prob_hbm, tgt_hbm, rew_hbm, out_hbm,
          tgt_v, idx_v, val_v, rew_v, acc_v, zero_v, izero_v, idx1_v,
          iz_sh, red_sh, lt_v, lt2_v,
          t0sem, t1sem, rsem,
          g0sem, g1sem, g2sem, g3sem, g4sem, g5sem, g6sem, g7sem):
    cid = lax.axis_index("c")
    sid = lax.axis_index("s")
    wid = sid * _NC + cid
    base = wid * _PW

    tcopy0 = pltpu.async_copy(tgt_hbm.at[pl.ds(base, _CHUNK)],
                              tgt_v.at[pl.ds(0, _CHUNK)], t0sem)
    tcopy1 = pltpu.async_copy(tgt_hbm.at[pl.ds(base + _CHUNK, _PW - _CHUNK)],
                              tgt_v.at[pl.ds(_CHUNK, _PW - _CHUNK)], t1sem)
    rcopy = pltpu.async_copy(rew_hbm.at[pl.ds(base, _PW)], rew_v, rsem)

    izero_v[...] = jnp.zeros((_L,), jnp.int32)
    pltpu.sync_copy(izero_v, iz_sh.at[sid])
    pltpu.sync_copy(iz_sh.at[sid].at[pl.ds(0, 1)], idx1_v)

    @pl.when(sid == 0)
    def _():
        zero_v[...] = jnp.zeros((_L,), jnp.float32)
        pltpu.sync_copy(zero_v, red_sh.at[0])

    lane = lax.iota(jnp.int32, _L)
    gsems = [g0sem, g1sem, g2sem, g3sem, g4sem, g5sem, g6sem, g7sem]
    gathers = []
    widr = lax.shift_left(wid, 13)
    tcopy0.wait()
    for j in range(_NCH):
        if j == 1:
            tcopy1.wait()
        for g in range(_CHUNK // _L):
            off = j * _CHUNK + g * _L
            rconst = ((off >> 7) << 10) + (off & 127)
            c = tgt_v[pl.ds(off, _L)]
            k = (
                lax.shift_left(lax.bitwise_and(c, jnp.int32(-8)), 14)
                + lax.shift_left(lax.bitwise_and(c, 7), 7)
                + (widr + (rconst + lane))
            )
            idx_v[j, pl.ds(g * _L, _L)] = k
        gathers.append(
            pltpu.async_copy(
                prob_hbm.at[idx_v.at[j]],
                val_v.at[pl.ds(j * _CHUNK, _CHUNK)],
                gsems[j],
            )
        )

    rcopy.wait()
    acc0 = jnp.zeros((_L,), jnp.float32)
    acc1 = jnp.zeros((_L,), jnp.float32)
    for j in range(_NCH):
        gathers[j].wait()
        for g in range(0, _CHUNK // _L, 2):
            t = j * (_CHUNK // _L) + g
            acc0 = acc0 + val_v[pl.ds(t * _L, _L)] * rew_v[pl.ds(t * _L, _L)]
            acc1 = acc1 + val_v[pl.ds((t + 1) * _L, _L)] * rew_v[pl.ds((t + 1) * _L, _L)]
    acc_v[0, :] = acc0 + acc1

    plsc.subcore_barrier()
    pltpu.sync_copy(acc_v, red_sh.at[idx1_v], add=True)
    plsc.subcore_barrier()

    @pl.when(sid == 0)
    def _():
        pltpu.sync_copy(red_sh.at[0], lt_v)
        tot = lt_v[...]
        lt2_v[pl.ds(_L, _L)] = tot
        for s in (8, 4, 2, 1):
            lt2_v[pl.ds(0, _L)] = tot
            tot = tot + lt2_v[pl.ds(s, _L)]
        zero_v[...] = tot * (-1.0 / _N)
        pltpu.sync_copy(zero_v.at[pl.ds(0, 1)], out_hbm)


_gather_loss = functools.partial(
    pl.kernel,
    out_type=jax.ShapeDtypeStruct((1,), jnp.float32),
    mesh=plsc.VectorSubcoreMesh(core_axis_name="c", subcore_axis_name="s",
                                num_cores=1),
    scratch_types=[
        pltpu.VMEM((_PW,), jnp.int32),
        pltpu.VMEM((_NCH, _CHUNK), jnp.int32),
        pltpu.VMEM((_PW,), jnp.float32),
        pltpu.VMEM((_PW,), jnp.float32),
        pltpu.VMEM((1, _L), jnp.float32),
        pltpu.VMEM((_L,), jnp.float32),
        pltpu.VMEM((_L,), jnp.int32),
        pltpu.VMEM((1,), jnp.int32),
        pltpu.VMEM_SHARED((_NS, _L), jnp.int32),
        pltpu.VMEM_SHARED((1, _L), jnp.float32),
        pltpu.VMEM((_L,), jnp.float32),
        pltpu.VMEM((2 * _L,), jnp.float32),
        pltpu.SemaphoreType.DMA,
        pltpu.SemaphoreType.DMA,
        pltpu.SemaphoreType.DMA,
        pltpu.SemaphoreType.DMA,
        pltpu.SemaphoreType.DMA,
        pltpu.SemaphoreType.DMA,
        pltpu.SemaphoreType.DMA,
        pltpu.SemaphoreType.DMA,
        pltpu.SemaphoreType.DMA,
        pltpu.SemaphoreType.DMA,
        pltpu.SemaphoreType.DMA,
    ],
)(_body)


def kernel(prob, targets, reward):
    pflat = prob.reshape(128, 128, 1250, 8).transpose(2, 0, 3, 1).reshape(-1)
    return jnp.reshape(_gather_loss(pflat, targets, reward), ())


# speedup vs baseline: 1.0788x; 1.0031x over previous
import functools

import jax
import jax.numpy as jnp
from jax import lax
from jax.experimental import pallas as pl
from jax.experimental.pallas import tpu as pltpu
from jax.experimental.pallas import tpu_sc as plsc

_N = 16384
_C = 10000
_NC = 1
_NS = 16
_NW = _NC * _NS
_PW = _N // _NW
_CHUNK = 128
_NCH = _PW // _CHUNK
_L = 16


def _body(prob_hbm, tgt_hbm, rew_hbm, out_hbm,
          tgt_v, idx_v, val_v, rew_v, acc_v, zero_v, izero_v, idx1_v,
          iz_sh, red_sh, lt_v, lt2_v,
          t0sem, t1sem, rsem, i0sem, i1sem,
          g0sem, g1sem, g2sem, g3sem, g4sem, g5sem, g6sem, g7sem):
    cid = lax.axis_index("c")
    sid = lax.axis_index("s")
    wid = sid * _NC + cid
    base = wid * _PW

    tcopy0 = pltpu.async_copy(tgt_hbm.at[pl.ds(base, _CHUNK)],
                              tgt_v.at[pl.ds(0, _CHUNK)], t0sem)
    tcopy1 = pltpu.async_copy(tgt_hbm.at[pl.ds(base + _CHUNK, _PW - _CHUNK)],
                              tgt_v.at[pl.ds(_CHUNK, _PW - _CHUNK)], t1sem)
    rcopy = pltpu.async_copy(rew_hbm.at[pl.ds(base, _PW)], rew_v, rsem)

    izero_v[...] = jnp.zeros((_L,), jnp.int32)
    izcopy = pltpu.async_copy(izero_v, iz_sh.at[sid], i0sem)

    @pl.when(sid == 0)
    def _():
        zero_v[...] = jnp.zeros((_L,), jnp.float32)
        pltpu.sync_copy(zero_v, red_sh.at[0])

    lane = lax.iota(jnp.int32, _L)
    gsems = [g0sem, g1sem, g2sem, g3sem, g4sem, g5sem, g6sem, g7sem]
    gathers = []
    widr = lax.shift_left(wid, 13)
    tcopy0.wait()
    icopy = None
    for j in range(_NCH):
        if j == 1:
            tcopy1.wait()
            izcopy.wait()
            icopy = pltpu.async_copy(iz_sh.at[sid].at[pl.ds(0, 1)],
                                     idx1_v, i1sem)
        for g in range(_CHUNK // _L):
            off = j * _CHUNK + g * _L
            rconst = ((off >> 7) << 10) + (off & 127)
            c = tgt_v[pl.ds(off, _L)]
            k = (
                lax.shift_left(lax.bitwise_and(c, jnp.int32(-8)), 14)
                + lax.shift_left(lax.bitwise_and(c, 7), 7)
                + (widr + (rconst + lane))
            )
            idx_v[j, pl.ds(g * _L, _L)] = k
        gathers.append(
            pltpu.async_copy(
                prob_hbm.at[idx_v.at[j]],
                val_v.at[pl.ds(j * _CHUNK, _CHUNK)],
                gsems[j],
            )
        )

    rcopy.wait()
    acc0 = jnp.zeros((_L,), jnp.float32)
    acc1 = jnp.zeros((_L,), jnp.float32)
    for j in range(_NCH):
        gathers[j].wait()
        for g in range(0, _CHUNK // _L, 2):
            t = j * (_CHUNK // _L) + g
            acc0 = acc0 + val_v[pl.ds(t * _L, _L)] * rew_v[pl.ds(t * _L, _L)]
            acc1 = acc1 + val_v[pl.ds((t + 1) * _L, _L)] * rew_v[pl.ds((t + 1) * _L, _L)]
    acc_v[0, :] = acc0 + acc1

    icopy.wait()
    plsc.subcore_barrier()
    pltpu.sync_copy(acc_v, red_sh.at[idx1_v], add=True)
    plsc.subcore_barrier()

    @pl.when(sid == 0)
    def _():
        pltpu.sync_copy(red_sh.at[0], lt_v)
        tot = lt_v[...]
        lt2_v[pl.ds(_L, _L)] = tot
        for s in (8, 4, 2, 1):
            lt2_v[pl.ds(0, _L)] = tot
            tot = tot + lt2_v[pl.ds(s, _L)]
        zero_v[...] = tot * (-1.0 / _N)
        pltpu.sync_copy(zero_v.at[pl.ds(0, 1)], out_hbm)


_gather_loss = functools.partial(
    pl.kernel,
    out_type=jax.ShapeDtypeStruct((1,), jnp.float32),
    mesh=plsc.VectorSubcoreMesh(core_axis_name="c", subcore_axis_name="s",
                                num_cores=1),
    scratch_types=[
        pltpu.VMEM((_PW,), jnp.int32),
        pltpu.VMEM((_NCH, _CHUNK), jnp.int32),
        pltpu.VMEM((_PW,), jnp.float32),
        pltpu.VMEM((_PW,), jnp.float32),
        pltpu.VMEM((1, _L), jnp.float32),
        pltpu.VMEM((_L,), jnp.float32),
        pltpu.VMEM((_L,), jnp.int32),
        pltpu.VMEM((1,), jnp.int32),
        pltpu.VMEM_SHARED((_NS, _L), jnp.int32),
        pltpu.VMEM_SHARED((1, _L), jnp.float32),
        pltpu.VMEM((_L,), jnp.float32),
        pltpu.VMEM((2 * _L,), jnp.float32),
        pltpu.SemaphoreType.DMA,
        pltpu.SemaphoreType.DMA,
        pltpu.SemaphoreType.DMA,
        pltpu.SemaphoreType.DMA,
        pltpu.SemaphoreType.DMA,
        pltpu.SemaphoreType.DMA,
        pltpu.SemaphoreType.DMA,
        pltpu.SemaphoreType.DMA,
        pltpu.SemaphoreType.DMA,
        pltpu.SemaphoreType.DMA,
        pltpu.SemaphoreType.DMA,
        pltpu.SemaphoreType.DMA,
        pltpu.SemaphoreType.DMA,
    ],
)(_body)


def kernel(prob, targets, reward):
    pflat = prob.reshape(128, 128, 1250, 8).transpose(2, 0, 3, 1).reshape(-1)
    return jnp.reshape(_gather_loss(pflat, targets, reward), ())
